# Initial kernel scaffold; baseline (speedup 1.0000x reference)
import jax
import jax.numpy as jnp
from jax.experimental import pallas as pl


def kernel(images):
    raise NotImplementedError("write your pallas kernel here")


# speedup vs baseline: 109.3559x; 109.3559x over previous
import jax
import jax.numpy as jnp
from jax import lax
from jax.experimental import pallas as pl
from jax.experimental.pallas import tpu as pltpu
from jax.experimental.pallas import tpu_sc as plsc

N_IMAGES = 32
H = W = 512
NPIX = H * W
WORDS = NPIX * 3
CHUNK = 49152
NCHUNK = WORDS // CHUNK
UNROLL = 4
NITER = CHUNK // (48 * UNROLL)
L = 16


def _equalize_body(img_hbm, out_hbm, buf, hist, lut, si0, si1, so0, so1):
    i32 = jnp.int32
    wid = lax.axis_index("s") * 2 + lax.axis_index("c")
    iota = lax.iota(i32, L)
    ones = jnp.broadcast_to(jnp.int32(1), (L,))
    choff = [((iota + 16 * k) % 3) * 256 for k in range(3)]
    sin = [si0, si1]
    sout = [so0, so1]

    def start_in(i):
        return pltpu.async_copy(
            img_hbm.at[wid, pl.ds(i * CHUNK, CHUNK)], buf.at[i % 2], sin[i % 2])

    def start_out(i):
        return pltpu.async_copy(
            buf.at[i % 2], out_hbm.at[wid, pl.ds(i * CHUNK, CHUNK)], sout[i % 2])

    def wait_in(i):
        pltpu.make_async_copy(
            img_hbm.at[wid, pl.ds(i * CHUNK, CHUNK)], buf.at[i % 2],
            sin[i % 2]).wait()

    def wait_out(i):
        pltpu.make_async_copy(
            buf.at[i % 2], out_hbm.at[wid, pl.ds(i * CHUNK, CHUNK)],
            sout[i % 2]).wait()

    zero = jnp.broadcast_to(jnp.int32(0), (L,))
    for j in range(768 // L):
        hist[pl.ds(j * L, L)] = zero

    start_in(0)
    for i in range(NCHUNK):
        if i + 1 < NCHUNK:
            start_in(i + 1)
        wait_in(i)
        slot = i % 2

        def hist_step(j, carry):
            base = j * (48 * UNROLL)
            for u in range(UNROLL):
                for k in range(3):
                    v = buf[slot, pl.ds(base + u * 48 + k * 16, L)]
                    plsc.addupdate_scatter(hist, [v + choff[k]], ones)
            return carry

        lax.fori_loop(0, NITER, hist_step, 0, unroll=False)

    for ch in range(3):
        hbase = ch * 256
        last = jnp.broadcast_to(jnp.int32(-1), (L,))
        for j in range(16):
            h = hist[pl.ds(hbase + j * L, L)]
            last = jnp.maximum(last, jnp.where(h != 0, iota + j * L, -1))
        last_idx = jnp.max(last)
        last_nz = plsc.load_gather(
            hist, [jnp.broadcast_to(hbase + last_idx, (L,))])
        step = lax.div(jnp.broadcast_to(jnp.int32(NPIX), (L,)) - last_nz, 255)
        step_zero = step == 0
        safe = jnp.where(step_zero, 1, step)
        half = lax.div(safe, 2)
        total = zero
        for j in range(16):
            h = hist[pl.ds(hbase + j * L, L)]
            inc = plsc.cumsum(h)
            excl = total + inc - h
            total = total + jnp.broadcast_to(jnp.max(inc), (L,))
            lutv = lax.div(excl + half, safe)
            lutv = jnp.minimum(jnp.maximum(lutv, 0), 255)
            lutv = jnp.where(step_zero, iota + j * L, lutv)
            lut[pl.ds(hbase + j * L, L)] = lutv

    start_in(0)
    for i in range(NCHUNK):
        if i >= 1:
            wait_out(i - 1)
        if i + 1 < NCHUNK:
            start_in(i + 1)
        wait_in(i)
        slot = i % 2

        def remap_step(j, carry):
            base = j * (48 * UNROLL)
            for u in range(UNROLL):
                for k in range(3):
                    off = base + u * 48 + k * 16
                    v = buf[slot, pl.ds(off, L)]
                    buf[slot, pl.ds(off, L)] = plsc.load_gather(
                        lut, [v + choff[k]])
            return carry

        lax.fori_loop(0, NITER, remap_step, 0, unroll=False)
        start_out(i)
    wait_out(NCHUNK - 1)


def kernel(images):
    flat = images.reshape(N_IMAGES, WORDS)
    run = pl.kernel(
        _equalize_body,
        out_type=jax.ShapeDtypeStruct((N_IMAGES, WORDS), jnp.int32),
        mesh=plsc.VectorSubcoreMesh(core_axis_name="c", subcore_axis_name="s",
                                    num_cores=2, num_subcores=16),
        compiler_params=pltpu.CompilerParams(needs_layout_passes=False),
        scratch_types=[
            pltpu.VMEM((2, CHUNK), jnp.int32),
            pltpu.VMEM((768,), jnp.int32),
            pltpu.VMEM((768,), jnp.int32),
            pltpu.SemaphoreType.DMA,
            pltpu.SemaphoreType.DMA,
            pltpu.SemaphoreType.DMA,
            pltpu.SemaphoreType.DMA,
        ],
    )
    return run(flat).reshape(images.shape)


# speedup vs baseline: 145.9728x; 1.3348x over previous
import jax
import jax.numpy as jnp
from jax import lax
from jax.experimental import pallas as pl
from jax.experimental.pallas import tpu as pltpu
from jax.experimental.pallas import tpu_sc as plsc

N_IMAGES = 32
H = W = 512
NPIX = H * W
WORDS = NPIX * 3
CHUNK = 49152
NCHUNK = WORDS // CHUNK
TRIPLES = CHUNK // 48
PLU = 8
L = 16


def _equalize_body(img_hbm, out_hbm, buf, hist, lut, si0, si1, so0, so1):
    i32 = jnp.int32
    wid = lax.axis_index("s") * 2 + lax.axis_index("c")
    iota = lax.iota(i32, L)
    ones = jnp.broadcast_to(jnp.int32(1), (L,))
    choff = [((iota + 16 * k) % 3) * 256 for k in range(3)]
    sin = [si0, si1]
    sout = [so0, so1]

    def start_in(i, s):
        return pltpu.async_copy(
            img_hbm.at[wid, pl.ds(i * CHUNK, CHUNK)], buf.at[s], sin[s])

    def start_out(i, s):
        return pltpu.async_copy(
            buf.at[s], out_hbm.at[wid, pl.ds(i * CHUNK, CHUNK)], sout[s])

    def wait_in(i, s):
        pltpu.make_async_copy(
            img_hbm.at[wid, pl.ds(i * CHUNK, CHUNK)], buf.at[s],
            sin[s]).wait()

    def wait_out(i, s):
        pltpu.make_async_copy(
            buf.at[s], out_hbm.at[wid, pl.ds(i * CHUNK, CHUNK)],
            sout[s]).wait()

    zero = jnp.broadcast_to(jnp.int32(0), (L,))
    for j in range(768 // L):
        hist[pl.ds(j * L, L)] = zero

    def hist_chunk(i, slot):
        @pl.when(i + 1 < NCHUNK)
        def _():
            start_in(i + 1, (slot + 1) % 2)
        wait_in(i, slot)

        @plsc.parallel_loop(0, TRIPLES, 1, unroll=PLU)
        def _hist_step(j):
            base = j * 48
            for k in range(3):
                v = buf[slot, pl.ds(base + k * 16, L)]
                plsc.addupdate_scatter(hist, [v + choff[k]], ones)

    start_in(0, 0)

    def hist_pair(p, carry):
        hist_chunk(2 * p, 0)
        hist_chunk(2 * p + 1, 1)
        return carry

    lax.fori_loop(0, NCHUNK // 2, hist_pair, 0)

    for ch in range(3):
        hbase = ch * 256
        last = jnp.broadcast_to(jnp.int32(-1), (L,))
        for j in range(16):
            h = hist[pl.ds(hbase + j * L, L)]
            last = jnp.maximum(last, jnp.where(h != 0, iota + j * L, -1))
        last_idx = jnp.max(last)
        last_nz = plsc.load_gather(
            hist, [jnp.broadcast_to(hbase + last_idx, (L,))])
        step = lax.div(jnp.broadcast_to(jnp.int32(NPIX), (L,)) - last_nz, 255)
        step_zero = step == 0
        safe = jnp.where(step_zero, 1, step)
        half = lax.div(safe, 2)
        total = zero
        for j in range(16):
            h = hist[pl.ds(hbase + j * L, L)]
            inc = plsc.cumsum(h)
            excl = total + inc - h
            total = total + jnp.broadcast_to(jnp.max(inc), (L,))
            lutv = lax.div(excl + half, safe)
            lutv = jnp.minimum(jnp.maximum(lutv, 0), 255)
            lutv = jnp.where(step_zero, iota + j * L, lutv)
            lut[pl.ds(hbase + j * L, L)] = lutv

    def remap_chunk(i, slot):
        @pl.when(i >= 1)
        def _():
            wait_out(i - 1, (slot + 1) % 2)

        @pl.when(i + 1 < NCHUNK)
        def _():
            start_in(i + 1, (slot + 1) % 2)
        wait_in(i, slot)

        @plsc.parallel_loop(0, TRIPLES, 1, unroll=PLU)
        def _remap_step(j):
            base = j * 48
            for k in range(3):
                off = base + k * 16
                v = buf[slot, pl.ds(off, L)]
                buf[slot, pl.ds(off, L)] = plsc.load_gather(
                    lut, [v + choff[k]])
        start_out(i, slot)

    start_in(0, 0)

    def remap_pair(p, carry):
        remap_chunk(2 * p, 0)
        remap_chunk(2 * p + 1, 1)
        return carry

    lax.fori_loop(0, NCHUNK // 2, remap_pair, 0)
    wait_out(NCHUNK - 1, (NCHUNK - 1) % 2)


def kernel(images):
    flat = images.reshape(N_IMAGES, WORDS)
    run = pl.kernel(
        _equalize_body,
        out_type=jax.ShapeDtypeStruct((N_IMAGES, WORDS), jnp.int32),
        mesh=plsc.VectorSubcoreMesh(core_axis_name="c", subcore_axis_name="s",
                                    num_cores=2, num_subcores=16),
        compiler_params=pltpu.CompilerParams(needs_layout_passes=False),
        scratch_types=[
            pltpu.VMEM((2, CHUNK), jnp.int32),
            pltpu.VMEM((768,), jnp.int32),
            pltpu.VMEM((768,), jnp.int32),
            pltpu.SemaphoreType.DMA,
            pltpu.SemaphoreType.DMA,
            pltpu.SemaphoreType.DMA,
            pltpu.SemaphoreType.DMA,
        ],
    )
    return run(flat).reshape(images.shape)
